# baseline (speedup 1.0000x reference)
import functools

import jax
import jax.numpy as jnp
from jax import lax
from jax.experimental import pallas as pl
from jax.experimental.pallas import tpu as pltpu
from jax.experimental.pallas import tpu_sc as plsc

DEG_EPS = 1e-4
EPS = 1e-7
LN_EPS = 1e-5

_NC = 2
_NS = 16
_NW = _NC * _NS


def _rsqrt(x):
    i = lax.bitcast_convert_type(x, jnp.int32)
    i = jnp.int32(0x5F3759DF) - lax.shift_right_arithmetic(i, 1)
    y = lax.bitcast_convert_type(i, jnp.float32)
    for _ in range(3):
        y = y * (1.5 - 0.5 * x * y * y)
    return y


def _geom_group(g, C, koff, src_v, dst_v, pv_v, gf_v):
    sl = pl.ds(koff + g * 16, 16)
    s16 = src_v[sl]
    d16 = dst_v[sl]

    s8 = lax.shift_left(s16, 2) + lax.shift_left(s16, 1)
    d8 = lax.shift_left(d16, 2) + lax.shift_left(d16, 1)

    def col(idx8, c):
        return plsc.load_gather(pv_v, [idx8 + jnp.int32(c)])

    pix, piy, piz = col(s8, 0), col(s8, 1), col(s8, 2)
    vix, viy, viz = col(s8, 3), col(s8, 4), col(s8, 5)
    pjx, pjy, pjz = col(d8, 0), col(d8, 1), col(d8, 2)
    vjx, vjy, vjz = col(d8, 3), col(d8, 4), col(d8, 5)

    rx, ry, rz = pjx - pix, pjy - piy, pjz - piz
    d2 = rx * rx + ry * ry + rz * rz
    dij = d2 * _rsqrt(d2)
    inv1 = 1.0 / (dij + EPS)
    e1x, e1y, e1z = rx * inv1, ry * inv1, rz * inv1

    vrx, vry, vrz = vjx - vix, vjy - viy, vjz - viz
    v_r = vrx * e1x + vry * e1y + vrz * e1z
    vpx = vrx - v_r * e1x
    vpy = vry - v_r * e1y
    vpz = vrz - v_r * e1z
    p2 = vpx * vpx + vpy * vpy + vpz * vpz
    pn = p2 * _rsqrt(p2)
    invp = 1.0 / (pn + EPS)
    e2vx, e2vy, e2vz = vpx * invp, vpy * invp, vpz * invp
    nd = pn > DEG_EPS

    fx0, fy0 = e1y, -e1x
    fn2 = fx0 * fx0 + fy0 * fy0
    fn = fn2 * _rsqrt(fn2)
    use_y = fn < DEG_EPS
    gx = jnp.where(use_y, -e1z, fx0)
    gy = jnp.where(use_y, jnp.float32(0.0), fy0)
    gz = jnp.where(use_y, e1x, jnp.float32(0.0))
    g2 = gx * gx + gy * gy + gz * gz
    gn = g2 * _rsqrt(g2)
    invg = 1.0 / (gn + EPS)
    e2fx, e2fy, e2fz = gx * invg, gy * invg, gz * invg

    e2x = jnp.where(nd, e2vx, e2fx)
    e2y = jnp.where(nd, e2vy, e2fy)
    e2z = jnp.where(nd, e2vz, e2fz)

    e3x = e1y * e2z - e1z * e2y
    e3y = e1z * e2x - e1x * e2z
    e3z = e1x * e2y - e1y * e2x

    v_t = vrx * e2x + vry * e2y + vrz * e2z
    v_b = vrx * e3x + vry * e3y + vrz * e3z
    q2 = vrx * vrx + vry * vry + vrz * vrz
    v_norm = q2 * _rsqrt(q2)

    maskf = jnp.where(s16 < d16, jnp.float32(1.0), jnp.float32(0.0))

    rows = (dij, v_r, v_t, v_b, v_norm,
            e1x * maskf, e1y * maskf, e1z * maskf,
            e2x * maskf, e2y * maskf, e2z * maskf,
            e3x * maskf, e3y * maskf, e3z * maskf,
            jnp.zeros((16,), jnp.float32), jnp.zeros((16,), jnp.float32))
    for r, val in enumerate(rows):
        gf_v[pl.ds(r * C + g * 16, 16)] = val


def _make_sc_front(N, D, E0, L, C):
    EW = L // _NW
    n = EW // C
    n_groups = C // 16
    mesh = plsc.VectorSubcoreMesh(core_axis_name="c", subcore_axis_name="s")

    @functools.partial(
        pl.kernel,
        out_type=(
            jax.ShapeDtypeStruct((L, D), jnp.float32),
            jax.ShapeDtypeStruct((L, D), jnp.float32),
            jax.ShapeDtypeStruct((16 * L,), jnp.float32),
        ),
        mesh=mesh,
        compiler_params=pltpu.CompilerParams(needs_layout_passes=False),
        scratch_types=[
            pltpu.VMEM((6 * N,), jnp.float32),
            pltpu.VMEM((EW,), jnp.int32),
            pltpu.VMEM((EW,), jnp.int32),
            pltpu.VMEM((C,), jnp.int32),
            pltpu.VMEM((C,), jnp.int32),
            pltpu.VMEM((C,), jnp.int32),
            pltpu.VMEM((C,), jnp.int32),
            pltpu.VMEM((C, D), jnp.float32),
            pltpu.VMEM((C, D), jnp.float32),
            pltpu.VMEM((C, D), jnp.float32),
            pltpu.VMEM((C, D), jnp.float32),
            pltpu.VMEM((16 * C,), jnp.float32),
            pltpu.VMEM((16 * C,), jnp.float32),
            pltpu.SemaphoreType.DMA,
            pltpu.SemaphoreType.DMA,
            pltpu.SemaphoreType.DMA,
            pltpu.SemaphoreType.DMA,
        ],
    )
    def sc_front(h_hbm, pv_hbm, srci_hbm, dsti_hbm, hsrc_o, hdst_o, gf_o,
                 pv_v, srca, dsta, sb0, db0, sb1, db1,
                 hs0, hd0, hs1, hd1, gf0, gf1,
                 gsem0, gsem1, wsem0, wsem1):
        wid = lax.axis_index("s") * _NC + lax.axis_index("c")
        base0 = E0 + wid * EW
        pltpu.sync_copy(pv_hbm, pv_v)
        pltpu.sync_copy(srci_hbm.at[pl.ds(base0, EW)], srca)
        pltpu.sync_copy(dsti_hbm.at[pl.ds(base0, EW)], dsta)

        def stage_idx(k, sb, db):
            off = k * C
            for j in range(C // 16):
                jj = pl.ds(j * 16, 16)
                sb[jj] = srca[pl.ds(off + j * 16, 16)]
                db[jj] = dsta[pl.ds(off + j * 16, 16)]

        stage_idx(0, sb0, db0)
        pltpu.async_copy(h_hbm.at[sb0], hs0, gsem0)
        pltpu.async_copy(h_hbm.at[db0], hd0, gsem0)

        bufs = ((sb0, db0, hs0, hd0, gf0, gsem0, wsem0),
                (sb1, db1, hs1, hd1, gf1, gsem1, wsem1))

        def body(k, _):
            def run(p):
                sb_p, db_p, hs_p, hd_p, gf_p, gsem_p, wsem_p = bufs[p]
                sb_q, db_q, hs_q, hd_q, gf_q, gsem_q, wsem_q = bufs[1 - p]
                base = wid * EW + k * C
                for g in range(n_groups):
                    _geom_group(g, C, k * C, srca, dsta, pv_v, gf_p)
                pltpu.make_async_copy(h_hbm.at[sb_p], hs_p, gsem_p).wait()
                pltpu.make_async_copy(h_hbm.at[db_p], hd_p, gsem_p).wait()

                @pl.when(k + 1 < n)
                def _():
                    @pl.when(k >= 1)
                    def _():
                        pltpu.make_async_copy(
                            hs_q, hsrc_o.at[pl.ds(0, C)], wsem_q).wait()
                        pltpu.make_async_copy(
                            hd_q, hdst_o.at[pl.ds(0, C)], wsem_q).wait()
                        pltpu.make_async_copy(
                            gf_q, gf_o.at[pl.ds(0, 16 * C)], wsem_q).wait()
                    stage_idx(k + 1, sb_q, db_q)
                    pltpu.async_copy(h_hbm.at[sb_q], hs_q, gsem_q)
                    pltpu.async_copy(h_hbm.at[db_q], hd_q, gsem_q)

                pltpu.async_copy(hs_p, hsrc_o.at[pl.ds(base, C)], wsem_p)
                pltpu.async_copy(hd_p, hdst_o.at[pl.ds(base, C)], wsem_p)
                pltpu.async_copy(gf_p, gf_o.at[pl.ds(base * 16, 16 * C)], wsem_p)

            p_is0 = lax.rem(k, 2) == 0
            pl.when(p_is0)(lambda: run(0))
            pl.when(jnp.logical_not(p_is0))(lambda: run(1))
            return 0

        lax.fori_loop(0, n, body, 0)
        for (_s, _d, hs_p, hd_p, gf_p, _g, wsem_p) in bufs:
            pltpu.make_async_copy(hs_p, hsrc_o.at[pl.ds(0, C)], wsem_p).wait()
            pltpu.make_async_copy(hd_p, hdst_o.at[pl.ds(0, C)], wsem_p).wait()
            pltpu.make_async_copy(gf_p, gf_o.at[pl.ds(0, 16 * C)], wsem_p).wait()

    return sc_front


def _make_sc_scatter(N, E, C):
    EW = E // _NW
    n_chunks = EW // C
    n_groups = C // 16
    N4 = 4 * N
    mesh = plsc.VectorSubcoreMesh(core_axis_name="c", subcore_axis_name="s")

    @functools.partial(
        pl.kernel,
        out_type=jax.ShapeDtypeStruct((_NW * N4,), jnp.float32),
        mesh=mesh,
        compiler_params=pltpu.CompilerParams(needs_layout_passes=False),
        scratch_types=[
            pltpu.VMEM((C,), jnp.int32),
            pltpu.VMEM((C,), jnp.int32),
            pltpu.VMEM((C,), jnp.float32),
            pltpu.VMEM((C,), jnp.float32),
            pltpu.VMEM((C,), jnp.float32),
            pltpu.VMEM((N4,), jnp.float32),
        ],
    )
    def sc2(ftf_hbm, srci_hbm, dsti_hbm, out_hbm,
            src_v, dst_v, fx_v, fy_v, fz_v, facc):
        wid = lax.axis_index("s") * _NC + lax.axis_index("c")

        def zero(i, _):
            facc[pl.ds(i * 16, 16)] = jnp.zeros((16,), jnp.float32)
            return 0

        lax.fori_loop(0, N4 // 16, zero, 0)

        def chunk(k, _):
            base = wid * EW + k * C
            pltpu.sync_copy(srci_hbm.at[pl.ds(base, C)], src_v)
            pltpu.sync_copy(dsti_hbm.at[pl.ds(base, C)], dst_v)
            pltpu.sync_copy(ftf_hbm.at[pl.ds(base, C)], fx_v)
            pltpu.sync_copy(ftf_hbm.at[pl.ds(E + base, C)], fy_v)
            pltpu.sync_copy(ftf_hbm.at[pl.ds(2 * E + base, C)], fz_v)
            for g in range(n_groups):
                sl = pl.ds(g * 16, 16)
                s16 = lax.shift_left(src_v[sl], 2)
                d16 = lax.shift_left(dst_v[sl], 2)
                fx = fx_v[sl]
                fy = fy_v[sl]
                fz = fz_v[sl]
                one = jnp.int32(1)
                two = jnp.int32(2)
                plsc.addupdate_scatter(facc, [d16], fx)
                plsc.addupdate_scatter(facc, [d16 + one], fy)
                plsc.addupdate_scatter(facc, [d16 + two], fz)
                plsc.addupdate_scatter(facc, [s16], -fx)
                plsc.addupdate_scatter(facc, [s16 + one], -fy)
                plsc.addupdate_scatter(facc, [s16 + two], -fz)
            return 0

        lax.fori_loop(0, n_chunks, chunk, 0)
        pltpu.sync_copy(facc, out_hbm.at[pl.ds(wid * N4, N4)])

    return sc2


def _edge_mlp_body(hsrc_ref, hdst_ref, gf_ref, w1s_ref, w1d_ref, w1g_ref,
                   b1_ref, g1_ref, be1_ref, w2_ref, b2c_ref, out_ref):
    a = hsrc_ref[...]
    b = hdst_ref[...]
    s = a + b
    d = jnp.abs(a - b)
    gfT = gf_ref[...]
    g8T = gfT[0:8, :]
    x = jnp.dot(s, w1s_ref[...], preferred_element_type=jnp.float32)
    x = x + jnp.dot(d, w1d_ref[...], preferred_element_type=jnp.float32)
    x = x + lax.dot_general(g8T, w1g_ref[...], (((0,), (0,)), ((), ())),
                            preferred_element_type=jnp.float32)
    x = x + b1_ref[...]
    mu = jnp.mean(x, axis=-1, keepdims=True)
    xc = x - mu
    var = jnp.mean(xc * xc, axis=-1, keepdims=True)
    x = xc * lax.rsqrt(var + LN_EPS) * g1_ref[...] + be1_ref[...]
    x = jnp.maximum(x, 0.0)
    a3 = lax.dot_general(w2_ref[...], x, (((1,), (1,)), ((), ())),
                         preferred_element_type=jnp.float32)
    a3 = a3 + b2c_ref[...]
    fx = (a3[0:1, :] * gfT[5:6, :] + a3[1:2, :] * gfT[8:9, :]
          + a3[2:3, :] * gfT[11:12, :])
    fy = (a3[0:1, :] * gfT[6:7, :] + a3[1:2, :] * gfT[9:10, :]
          + a3[2:3, :] * gfT[12:13, :])
    fz = (a3[0:1, :] * gfT[7:8, :] + a3[1:2, :] * gfT[10:11, :]
          + a3[2:3, :] * gfT[13:14, :])
    z5 = jnp.zeros((5, fx.shape[1]), jnp.float32)
    out_ref[...] = jnp.concatenate([fx, fy, fz, z5], axis=0)


def _reduce_body(fp_ref, out_ref):
    out_ref[...] = jnp.sum(fp_ref[...], axis=0, keepdims=True)


def _node_mlp_body(h_ref, f4_ref, w1h_ref, w1f_ref, b1_ref, g1_ref, be1_ref,
                   w2_ref, b2_ref, out_ref):
    h = h_ref[...]
    x = jnp.dot(h, w1h_ref[...], preferred_element_type=jnp.float32)
    x = x + jnp.dot(f4_ref[...], w1f_ref[...], preferred_element_type=jnp.float32)
    x = x + b1_ref[...]
    mu = jnp.mean(x, axis=-1, keepdims=True)
    xc = x - mu
    var = jnp.mean(xc * xc, axis=-1, keepdims=True)
    x = xc * lax.rsqrt(var + LN_EPS) * g1_ref[...] + be1_ref[...]
    x = jnp.maximum(x, 0.0)
    o = jnp.dot(x, w2_ref[...], preferred_element_type=jnp.float32)
    out_ref[...] = o + b2_ref[...] + h


def kernel(h, edge_index, pos, vel, W1f, b1f, g1f, be1f, W2f, b2f,
           W1n, b1n, g1n, be1n, W2n, b2n):
    N, D = h.shape
    E = edge_index.shape[1]
    H = W1f.shape[0]
    ei = edge_index.astype(jnp.int32)
    srci = ei[0]
    dsti = ei[1]

    CG = 80
    LA = 163840
    LB = E - LA
    pv6 = jnp.concatenate([pos, vel], axis=1).reshape(6 * N)
    halves = []
    for (e0, ln) in ((0, LA), (LA, LB)):
        fr = _make_sc_front(N, D, e0, ln, CG)
        hsrc_h, hdst_h, gfflat_h = fr(h, pv6, srci, dsti)
        gf_h = gfflat_h.reshape(ln // CG, 16, CG).transpose(1, 0, 2).reshape(16, ln)
        halves.append((ln, hsrc_h, hdst_h, gf_h))

    W1g8T = jnp.concatenate(
        [W1f[:, 0:5].T, jnp.zeros((3, H), jnp.float32)], axis=0)
    W1sT = W1f[:, 5:5 + D].T
    W1dT = W1f[:, 5 + D:5 + 2 * D].T
    b2c = b2f.reshape(3, 1)
    BE = 1280
    forces = []
    for (ln, hsrc_h, hdst_h, gf_h) in halves:
        forces.append(pl.pallas_call(
            _edge_mlp_body,
            grid=(ln // BE,),
            in_specs=[
                pl.BlockSpec((BE, D), lambda i: (i, 0)),
                pl.BlockSpec((BE, D), lambda i: (i, 0)),
                pl.BlockSpec((16, BE), lambda i: (0, i)),
                pl.BlockSpec((D, H), lambda i: (0, 0)),
                pl.BlockSpec((D, H), lambda i: (0, 0)),
                pl.BlockSpec((8, H), lambda i: (0, 0)),
                pl.BlockSpec((1, H), lambda i: (0, 0)),
                pl.BlockSpec((1, H), lambda i: (0, 0)),
                pl.BlockSpec((1, H), lambda i: (0, 0)),
                pl.BlockSpec((3, H), lambda i: (0, 0)),
                pl.BlockSpec((3, 1), lambda i: (0, 0)),
            ],
            out_specs=pl.BlockSpec((8, BE), lambda i: (0, i)),
            out_shape=jax.ShapeDtypeStruct((8, ln), jnp.float32),
        )(hsrc_h, hdst_h, gf_h, W1sT, W1dT, W1g8T, b1f.reshape(1, H),
          g1f.reshape(1, H), be1f.reshape(1, H), W2f, b2c))

    forceT = jnp.concatenate(forces, axis=1)

    sc2 = _make_sc_scatter(N, E, C=2000)
    Fp = sc2(forceT.reshape(8 * E), srci, dsti)

    N4 = 4 * N
    F4flat = pl.pallas_call(
        _reduce_body,
        out_shape=jax.ShapeDtypeStruct((1, N4), jnp.float32),
    )(Fp.reshape(_NW, N4))
    F4 = F4flat.reshape(N, 4)

    W1nhT = W1n[:, :D].T
    W1nF4T = jnp.concatenate(
        [W1n[:, D:].T, jnp.zeros((1, H), jnp.float32)], axis=0)
    BN = 2000
    h_new = pl.pallas_call(
        _node_mlp_body,
        grid=(N // BN,),
        in_specs=[
            pl.BlockSpec((BN, D), lambda i: (i, 0)),
            pl.BlockSpec((BN, 4), lambda i: (i, 0)),
            pl.BlockSpec((D, H), lambda i: (0, 0)),
            pl.BlockSpec((4, H), lambda i: (0, 0)),
            pl.BlockSpec((1, H), lambda i: (0, 0)),
            pl.BlockSpec((1, H), lambda i: (0, 0)),
            pl.BlockSpec((1, H), lambda i: (0, 0)),
            pl.BlockSpec((H, D), lambda i: (0, 0)),
            pl.BlockSpec((1, D), lambda i: (0, 0)),
        ],
        out_specs=pl.BlockSpec((BN, D), lambda i: (i, 0)),
        out_shape=jax.ShapeDtypeStruct((N, D), jnp.float32),
    )(h, F4, W1nhT, W1nF4T, b1n.reshape(1, H), g1n.reshape(1, H),
      be1n.reshape(1, H), W2n.T, b2n.reshape(1, D))
    return h_new


# speedup vs baseline: 1.0631x; 1.0631x over previous
import functools

import jax
import jax.numpy as jnp
from jax import lax
from jax.experimental import pallas as pl
from jax.experimental.pallas import tpu as pltpu
from jax.experimental.pallas import tpu_sc as plsc

DEG_EPS = 1e-4
EPS = 1e-7
LN_EPS = 1e-5

_NC = 2
_NS = 16
_NW = _NC * _NS


def _rsqrt(x):
    i = lax.bitcast_convert_type(x, jnp.int32)
    i = jnp.int32(0x5F3759DF) - lax.shift_right_arithmetic(i, 1)
    y = lax.bitcast_convert_type(i, jnp.float32)
    for _ in range(3):
        y = y * (1.5 - 0.5 * x * y * y)
    return y


def _geom_group(g, C, koff, src_v, dst_v, pv_v, gf_v):
    sl = pl.ds(koff + g * 16, 16)
    s16 = src_v[sl]
    d16 = dst_v[sl]

    s8 = lax.shift_left(s16, 2) + lax.shift_left(s16, 1)
    d8 = lax.shift_left(d16, 2) + lax.shift_left(d16, 1)

    def col(idx8, c):
        return plsc.load_gather(pv_v, [idx8 + jnp.int32(c)])

    pix, piy, piz = col(s8, 0), col(s8, 1), col(s8, 2)
    vix, viy, viz = col(s8, 3), col(s8, 4), col(s8, 5)
    pjx, pjy, pjz = col(d8, 0), col(d8, 1), col(d8, 2)
    vjx, vjy, vjz = col(d8, 3), col(d8, 4), col(d8, 5)

    rx, ry, rz = pjx - pix, pjy - piy, pjz - piz
    d2 = rx * rx + ry * ry + rz * rz
    dij = d2 * _rsqrt(d2)
    inv1 = 1.0 / (dij + EPS)
    e1x, e1y, e1z = rx * inv1, ry * inv1, rz * inv1

    vrx, vry, vrz = vjx - vix, vjy - viy, vjz - viz
    v_r = vrx * e1x + vry * e1y + vrz * e1z
    vpx = vrx - v_r * e1x
    vpy = vry - v_r * e1y
    vpz = vrz - v_r * e1z
    p2 = vpx * vpx + vpy * vpy + vpz * vpz
    pn = p2 * _rsqrt(p2)
    invp = 1.0 / (pn + EPS)
    e2vx, e2vy, e2vz = vpx * invp, vpy * invp, vpz * invp
    nd = pn > DEG_EPS

    fx0, fy0 = e1y, -e1x
    fn2 = fx0 * fx0 + fy0 * fy0
    fn = fn2 * _rsqrt(fn2)
    use_y = fn < DEG_EPS
    gx = jnp.where(use_y, -e1z, fx0)
    gy = jnp.where(use_y, jnp.float32(0.0), fy0)
    gz = jnp.where(use_y, e1x, jnp.float32(0.0))
    g2 = gx * gx + gy * gy + gz * gz
    gn = g2 * _rsqrt(g2)
    invg = 1.0 / (gn + EPS)
    e2fx, e2fy, e2fz = gx * invg, gy * invg, gz * invg

    e2x = jnp.where(nd, e2vx, e2fx)
    e2y = jnp.where(nd, e2vy, e2fy)
    e2z = jnp.where(nd, e2vz, e2fz)

    e3x = e1y * e2z - e1z * e2y
    e3y = e1z * e2x - e1x * e2z
    e3z = e1x * e2y - e1y * e2x

    v_t = vrx * e2x + vry * e2y + vrz * e2z
    v_b = vrx * e3x + vry * e3y + vrz * e3z
    q2 = vrx * vrx + vry * vry + vrz * vrz
    v_norm = q2 * _rsqrt(q2)

    maskf = jnp.where(s16 < d16, jnp.float32(1.0), jnp.float32(0.0))

    rows = (dij, v_r, v_t, v_b, v_norm,
            e1x * maskf, e1y * maskf, e1z * maskf,
            e2x * maskf, e2y * maskf, e2z * maskf,
            e3x * maskf, e3y * maskf, e3z * maskf,
            jnp.zeros((16,), jnp.float32), jnp.zeros((16,), jnp.float32))
    for r, val in enumerate(rows):
        gf_v[pl.ds(r * C + g * 16, 16)] = val


def _make_sc_front(N, D, E, C):
    EW = E // _NW
    n = EW // C
    n_groups = C // 16
    mesh = plsc.VectorSubcoreMesh(core_axis_name="c", subcore_axis_name="s")

    @functools.partial(
        pl.kernel,
        out_type=(
            jax.ShapeDtypeStruct((E, D), jnp.float32),
            jax.ShapeDtypeStruct((E, D), jnp.float32),
            jax.ShapeDtypeStruct((16 * E,), jnp.float32),
        ),
        mesh=mesh,
        compiler_params=pltpu.CompilerParams(needs_layout_passes=False),
        scratch_types=[
            pltpu.VMEM((6 * N,), jnp.float32),
            pltpu.VMEM((EW,), jnp.int32),
            pltpu.VMEM((EW,), jnp.int32),
            pltpu.VMEM((C,), jnp.int32),
            pltpu.VMEM((C,), jnp.int32),
            pltpu.VMEM((C,), jnp.int32),
            pltpu.VMEM((C,), jnp.int32),
            pltpu.VMEM((C, D), jnp.float32),
            pltpu.VMEM((C, D), jnp.float32),
            pltpu.VMEM((C, D), jnp.float32),
            pltpu.VMEM((C, D), jnp.float32),
            pltpu.VMEM((16 * C,), jnp.float32),
            pltpu.VMEM((16 * C,), jnp.float32),
            pltpu.SemaphoreType.DMA,
            pltpu.SemaphoreType.DMA,
            pltpu.SemaphoreType.DMA,
            pltpu.SemaphoreType.DMA,
        ],
    )
    def sc_front(h_hbm, pv_hbm, srci_hbm, dsti_hbm, hsrc_o, hdst_o, gf_o,
                 pv_v, srca, dsta, sb0, db0, sb1, db1,
                 hs0, hd0, hs1, hd1, gf0, gf1,
                 gsem0, gsem1, wsem0, wsem1):
        wid = lax.axis_index("s") * _NC + lax.axis_index("c")
        base0 = wid * EW
        pltpu.sync_copy(pv_hbm, pv_v)
        pltpu.sync_copy(srci_hbm.at[pl.ds(base0, EW)], srca)
        pltpu.sync_copy(dsti_hbm.at[pl.ds(base0, EW)], dsta)

        def stage_idx(k, sb, db):
            off = k * C
            for j in range(C // 16):
                jj = pl.ds(j * 16, 16)
                sb[jj] = srca[pl.ds(off + j * 16, 16)]
                db[jj] = dsta[pl.ds(off + j * 16, 16)]

        stage_idx(0, sb0, db0)
        pltpu.async_copy(h_hbm.at[sb0], hs0, gsem0)
        pltpu.async_copy(h_hbm.at[db0], hd0, gsem0)

        bufs = ((sb0, db0, hs0, hd0, gf0, gsem0, wsem0),
                (sb1, db1, hs1, hd1, gf1, gsem1, wsem1))

        def body(k, _):
            def run(p):
                sb_p, db_p, hs_p, hd_p, gf_p, gsem_p, wsem_p = bufs[p]
                sb_q, db_q, hs_q, hd_q, gf_q, gsem_q, wsem_q = bufs[1 - p]
                base = wid * EW + k * C
                for g in range(n_groups):
                    _geom_group(g, C, k * C, srca, dsta, pv_v, gf_p)
                pltpu.make_async_copy(h_hbm.at[sb_p], hs_p, gsem_p).wait()
                pltpu.make_async_copy(h_hbm.at[db_p], hd_p, gsem_p).wait()

                @pl.when(k + 1 < n)
                def _():
                    @pl.when(k >= 1)
                    def _():
                        pltpu.make_async_copy(
                            hs_q, hsrc_o.at[pl.ds(0, C)], wsem_q).wait()
                        pltpu.make_async_copy(
                            hd_q, hdst_o.at[pl.ds(0, C)], wsem_q).wait()
                        pltpu.make_async_copy(
                            gf_q, gf_o.at[pl.ds(0, 16 * C)], wsem_q).wait()
                    stage_idx(k + 1, sb_q, db_q)
                    pltpu.async_copy(h_hbm.at[sb_q], hs_q, gsem_q)
                    pltpu.async_copy(h_hbm.at[db_q], hd_q, gsem_q)

                pltpu.async_copy(hs_p, hsrc_o.at[pl.ds(base, C)], wsem_p)
                pltpu.async_copy(hd_p, hdst_o.at[pl.ds(base, C)], wsem_p)
                pltpu.async_copy(gf_p, gf_o.at[pl.ds(base * 16, 16 * C)], wsem_p)

            p_is0 = lax.rem(k, 2) == 0
            pl.when(p_is0)(lambda: run(0))
            pl.when(jnp.logical_not(p_is0))(lambda: run(1))
            return 0

        lax.fori_loop(0, n, body, 0)
        for (_s, _d, hs_p, hd_p, gf_p, _g, wsem_p) in bufs:
            pltpu.make_async_copy(hs_p, hsrc_o.at[pl.ds(0, C)], wsem_p).wait()
            pltpu.make_async_copy(hd_p, hdst_o.at[pl.ds(0, C)], wsem_p).wait()
            pltpu.make_async_copy(gf_p, gf_o.at[pl.ds(0, 16 * C)], wsem_p).wait()

    return sc_front


def _make_sc_scatter(N, E, C):
    EW = E // _NW
    n_chunks = EW // C
    n_groups = C // 16
    N4 = 4 * N
    mesh = plsc.VectorSubcoreMesh(core_axis_name="c", subcore_axis_name="s")

    @functools.partial(
        pl.kernel,
        out_type=jax.ShapeDtypeStruct((_NW * N4,), jnp.float32),
        mesh=mesh,
        compiler_params=pltpu.CompilerParams(needs_layout_passes=False),
        scratch_types=[
            pltpu.VMEM((C,), jnp.int32),
            pltpu.VMEM((C,), jnp.int32),
            pltpu.VMEM((C,), jnp.float32),
            pltpu.VMEM((C,), jnp.float32),
            pltpu.VMEM((C,), jnp.float32),
            pltpu.VMEM((N4,), jnp.float32),
        ],
    )
    def sc2(ftf_hbm, srci_hbm, dsti_hbm, out_hbm,
            src_v, dst_v, fx_v, fy_v, fz_v, facc):
        wid = lax.axis_index("s") * _NC + lax.axis_index("c")

        def zero(i, _):
            facc[pl.ds(i * 16, 16)] = jnp.zeros((16,), jnp.float32)
            return 0

        lax.fori_loop(0, N4 // 16, zero, 0)

        def chunk(k, _):
            base = wid * EW + k * C
            pltpu.sync_copy(srci_hbm.at[pl.ds(base, C)], src_v)
            pltpu.sync_copy(dsti_hbm.at[pl.ds(base, C)], dst_v)
            pltpu.sync_copy(ftf_hbm.at[pl.ds(base, C)], fx_v)
            pltpu.sync_copy(ftf_hbm.at[pl.ds(E + base, C)], fy_v)
            pltpu.sync_copy(ftf_hbm.at[pl.ds(2 * E + base, C)], fz_v)
            for g in range(n_groups):
                sl = pl.ds(g * 16, 16)
                s16 = lax.shift_left(src_v[sl], 2)
                d16 = lax.shift_left(dst_v[sl], 2)
                fx = fx_v[sl]
                fy = fy_v[sl]
                fz = fz_v[sl]
                one = jnp.int32(1)
                two = jnp.int32(2)
                plsc.addupdate_scatter(facc, [d16], fx)
                plsc.addupdate_scatter(facc, [d16 + one], fy)
                plsc.addupdate_scatter(facc, [d16 + two], fz)
                plsc.addupdate_scatter(facc, [s16], -fx)
                plsc.addupdate_scatter(facc, [s16 + one], -fy)
                plsc.addupdate_scatter(facc, [s16 + two], -fz)
            return 0

        lax.fori_loop(0, n_chunks, chunk, 0)
        pltpu.sync_copy(facc, out_hbm.at[pl.ds(wid * N4, N4)])

    return sc2


def _edge_mlp_body(hsrc_ref, hdst_ref, gf_ref, w1s_ref, w1d_ref, w1g_ref,
                   b1_ref, g1_ref, be1_ref, w2_ref, b2c_ref, out_ref):
    a = hsrc_ref[...]
    b = hdst_ref[...]
    s = a + b
    d = jnp.abs(a - b)
    gfT = gf_ref[...]
    g8T = gfT[0:8, :]
    x = jnp.dot(s, w1s_ref[...], preferred_element_type=jnp.float32)
    x = x + jnp.dot(d, w1d_ref[...], preferred_element_type=jnp.float32)
    x = x + lax.dot_general(g8T, w1g_ref[...], (((0,), (0,)), ((), ())),
                            preferred_element_type=jnp.float32)
    x = x + b1_ref[...]
    jm = jnp.full((x.shape[1], x.shape[1]), 1.0 / x.shape[1], jnp.float32)
    mu = jnp.dot(x, jm, preferred_element_type=jnp.float32)
    msq = jnp.dot(x * x, jm, preferred_element_type=jnp.float32)
    var = msq - mu * mu
    x = (x - mu) * lax.rsqrt(var + LN_EPS) * g1_ref[...] + be1_ref[...]
    x = jnp.maximum(x, 0.0)
    a3 = lax.dot_general(w2_ref[...], x, (((1,), (1,)), ((), ())),
                         preferred_element_type=jnp.float32)
    a3 = a3 + b2c_ref[...]
    fx = (a3[0:1, :] * gfT[5:6, :] + a3[1:2, :] * gfT[8:9, :]
          + a3[2:3, :] * gfT[11:12, :])
    fy = (a3[0:1, :] * gfT[6:7, :] + a3[1:2, :] * gfT[9:10, :]
          + a3[2:3, :] * gfT[12:13, :])
    fz = (a3[0:1, :] * gfT[7:8, :] + a3[1:2, :] * gfT[10:11, :]
          + a3[2:3, :] * gfT[13:14, :])
    z5 = jnp.zeros((5, fx.shape[1]), jnp.float32)
    out_ref[...] = jnp.concatenate([fx, fy, fz, z5], axis=0)


def _reduce_body(fp_ref, out_ref):
    out_ref[...] = jnp.sum(fp_ref[...], axis=0, keepdims=True)


def _node_mlp_body(h_ref, f4_ref, w1h_ref, w1f_ref, b1_ref, g1_ref, be1_ref,
                   w2_ref, b2_ref, out_ref):
    h = h_ref[...]
    x = jnp.dot(h, w1h_ref[...], preferred_element_type=jnp.float32)
    x = x + jnp.dot(f4_ref[...], w1f_ref[...], preferred_element_type=jnp.float32)
    x = x + b1_ref[...]
    mu = jnp.mean(x, axis=-1, keepdims=True)
    xc = x - mu
    var = jnp.mean(xc * xc, axis=-1, keepdims=True)
    x = xc * lax.rsqrt(var + LN_EPS) * g1_ref[...] + be1_ref[...]
    x = jnp.maximum(x, 0.0)
    o = jnp.dot(x, w2_ref[...], preferred_element_type=jnp.float32)
    out_ref[...] = o + b2_ref[...] + h


def kernel(h, edge_index, pos, vel, W1f, b1f, g1f, be1f, W2f, b2f,
           W1n, b1n, g1n, be1n, W2n, b2n):
    N, D = h.shape
    E = edge_index.shape[1]
    H = W1f.shape[0]
    ei = edge_index.astype(jnp.int32)
    srci = ei[0]
    dsti = ei[1]

    CG = 80
    sc_front = _make_sc_front(N, D, E, CG)
    pv6 = jnp.concatenate([pos, vel], axis=1).reshape(6 * N)
    hsrc, hdst, gfflat = sc_front(h, pv6, srci, dsti)
    n_chunks_total = E // CG
    gf = gfflat.reshape(n_chunks_total, 16, CG).transpose(1, 0, 2).reshape(16, E)

    W1g8T = jnp.concatenate(
        [W1f[:, 0:5].T, jnp.zeros((3, H), jnp.float32)], axis=0)
    W1sT = W1f[:, 5:5 + D].T
    W1dT = W1f[:, 5 + D:5 + 2 * D].T
    b2c = b2f.reshape(3, 1)
    BE = 2560
    grid_e = E // BE
    forceT = pl.pallas_call(
        _edge_mlp_body,
        grid=(grid_e,),
        in_specs=[
            pl.BlockSpec((BE, D), lambda i: (i, 0)),
            pl.BlockSpec((BE, D), lambda i: (i, 0)),
            pl.BlockSpec((16, BE), lambda i: (0, i)),
            pl.BlockSpec((D, H), lambda i: (0, 0)),
            pl.BlockSpec((D, H), lambda i: (0, 0)),
            pl.BlockSpec((8, H), lambda i: (0, 0)),
            pl.BlockSpec((1, H), lambda i: (0, 0)),
            pl.BlockSpec((1, H), lambda i: (0, 0)),
            pl.BlockSpec((1, H), lambda i: (0, 0)),
            pl.BlockSpec((3, H), lambda i: (0, 0)),
            pl.BlockSpec((3, 1), lambda i: (0, 0)),
        ],
        out_specs=pl.BlockSpec((8, BE), lambda i: (0, i)),
        out_shape=jax.ShapeDtypeStruct((8, E), jnp.float32),
    )(hsrc, hdst, gf, W1sT, W1dT, W1g8T, b1f.reshape(1, H),
      g1f.reshape(1, H), be1f.reshape(1, H), W2f, b2c)

    sc2 = _make_sc_scatter(N, E, C=2000)
    Fp = sc2(forceT.reshape(8 * E), srci, dsti)

    N4 = 4 * N
    F4flat = pl.pallas_call(
        _reduce_body,
        out_shape=jax.ShapeDtypeStruct((1, N4), jnp.float32),
    )(Fp.reshape(_NW, N4))
    F4 = F4flat.reshape(N, 4)

    W1nhT = W1n[:, :D].T
    W1nF4T = jnp.concatenate(
        [W1n[:, D:].T, jnp.zeros((1, H), jnp.float32)], axis=0)
    BN = 2000
    h_new = pl.pallas_call(
        _node_mlp_body,
        grid=(N // BN,),
        in_specs=[
            pl.BlockSpec((BN, D), lambda i: (i, 0)),
            pl.BlockSpec((BN, 4), lambda i: (i, 0)),
            pl.BlockSpec((D, H), lambda i: (0, 0)),
            pl.BlockSpec((4, H), lambda i: (0, 0)),
            pl.BlockSpec((1, H), lambda i: (0, 0)),
            pl.BlockSpec((1, H), lambda i: (0, 0)),
            pl.BlockSpec((1, H), lambda i: (0, 0)),
            pl.BlockSpec((H, D), lambda i: (0, 0)),
            pl.BlockSpec((1, D), lambda i: (0, 0)),
        ],
        out_specs=pl.BlockSpec((BN, D), lambda i: (i, 0)),
        out_shape=jax.ShapeDtypeStruct((N, D), jnp.float32),
    )(h, F4, W1nhT, W1nF4T, b1n.reshape(1, H), g1n.reshape(1, H),
      be1n.reshape(1, H), W2n.T, b2n.reshape(1, D))
    return h_new
